# Initial kernel scaffold; baseline (speedup 1.0000x reference)
import jax
import jax.numpy as jnp
from jax.experimental import pallas as pl


def kernel(hidden_states, w1, w2, topk_weights, topk_ids):
    raise NotImplementedError("write your pallas kernel here")


# speedup vs baseline: 1.0839x; 1.0839x over previous
import functools

import jax
import jax.numpy as jnp
from jax import lax
from jax.experimental import pallas as pl
from jax.experimental.pallas import tpu as pltpu
from jax.experimental.pallas import tpu_sc as plsc

E = 8
K = 2
T = 2048
D = 1024
FF = 2048

BT = 256
N = T * K
NB = (N + E * (BT - 1) + BT - 1) // BT
P = NB * BT

NC = 2
NS = 16
NW = NC * NS

ROWS_W = P // NW
CH = 64
NCH = ROWS_W // CH

TOK_W = T // NW
FCH = 32
NFCH = TOK_W // FCH

@functools.cache
def _build_sc_kernels():
    mesh = plsc.VectorSubcoreMesh(core_axis_name="c", subcore_axis_name="s")

    @functools.partial(
        pl.kernel,
        out_type=jax.ShapeDtypeStruct((P, D), jnp.float32),
        mesh=mesh,
        scratch_types=[
            pltpu.VMEM((NCH, CH), jnp.int32),
            pltpu.VMEM((CH, D), jnp.float32),
            pltpu.SemaphoreType.DMA,
        ],
    )
    def sc_dispatch(hs_hbm, tok_hbm, out_hbm, idx_v, rows_v, sem):
        wid = lax.axis_index("s") * NC + lax.axis_index("c")
        base = wid * ROWS_W
        pltpu.sync_copy(tok_hbm.at[wid], idx_v)
        for c in range(NCH):
            pltpu.async_copy(hs_hbm.at[idx_v.at[c]], rows_v, sem).wait()
            pltpu.sync_copy(rows_v, out_hbm.at[pl.ds(base + c * CH, CH)])

    @functools.partial(
        pl.kernel,
        out_type=jax.ShapeDtypeStruct((T, D), jnp.float32),
        mesh=mesh,
        scratch_types=[
            pltpu.VMEM((NFCH, FCH), jnp.int32),
            pltpu.VMEM((NFCH, FCH), jnp.int32),
            pltpu.VMEM((FCH, D), jnp.float32),
            pltpu.VMEM((FCH, D), jnp.float32),
            pltpu.SemaphoreType.DMA,
        ],
    )
    def sc_finalize(y_hbm, s0_hbm, s1_hbm, out_hbm, idx0_v, idx1_v, a_v, b_v, sem):
        wid = lax.axis_index("s") * NC + lax.axis_index("c")
        base = wid * TOK_W
        pltpu.sync_copy(s0_hbm.at[wid], idx0_v)
        pltpu.sync_copy(s1_hbm.at[wid], idx1_v)
        steps = D // 16
        for c in range(NFCH):
            pltpu.async_copy(y_hbm.at[idx0_v.at[c]], a_v, sem).wait()
            pltpu.async_copy(y_hbm.at[idx1_v.at[c]], b_v, sem).wait()

            def _add(i, _):
                r = i // steps
                col = (i % steps) * 16
                a_v[r, pl.ds(col, 16)] = (a_v[r, pl.ds(col, 16)]
                                          + b_v[r, pl.ds(col, 16)])
                return 0

            lax.fori_loop(0, FCH * steps, _add, 0)
            pltpu.sync_copy(a_v, out_hbm.at[pl.ds(base + c * FCH, FCH)])

    return sc_dispatch, sc_finalize


def _tc_moe_body(be_ref, x_ref, w1_ref, w2_ref, sw_ref, y_ref):
    x = x_ref[...]
    w1b = w1_ref[0]
    g = lax.dot_general(x, w1b[:FF], (((1,), (1,)), ((), ())),
                        preferred_element_type=jnp.float32)
    u = lax.dot_general(x, w1b[FF:], (((1,), (1,)), ((), ())),
                        preferred_element_type=jnp.float32)
    h = g * lax.logistic(g) * u
    y = lax.dot_general(h, w2_ref[0], (((1,), (1,)), ((), ())),
                        preferred_element_type=jnp.float32)
    y_ref[...] = y * sw_ref[...]


_tc_moe = pl.pallas_call(
    _tc_moe_body,
    grid_spec=pltpu.PrefetchScalarGridSpec(
        num_scalar_prefetch=1,
        grid=(NB,),
        in_specs=[
            pl.BlockSpec((BT, D), lambda i, be: (i, 0)),
            pl.BlockSpec((1, 2 * FF, D), lambda i, be: (be[i], 0, 0)),
            pl.BlockSpec((1, D, FF), lambda i, be: (be[i], 0, 0)),
            pl.BlockSpec((BT, 1), lambda i, be: (i, 0)),
        ],
        out_specs=pl.BlockSpec((BT, D), lambda i, be: (i, 0)),
    ),
    out_shape=jax.ShapeDtypeStruct((P, D), jnp.float32),
)


def kernel(hidden_states, w1, w2, topk_weights, topk_ids):
    flat_e = topk_ids.reshape(-1).astype(jnp.int32)
    order = jnp.argsort(flat_e, stable=True)
    sorted_e = flat_e[order]
    counts = jnp.bincount(flat_e, length=E)
    raw_cum = jnp.cumsum(counts)
    raw_off = raw_cum - counts
    pad_counts = ((counts + BT - 1) // BT) * BT
    pad_cum = jnp.cumsum(pad_counts)
    pad_off = pad_cum - pad_counts
    slot = (pad_off[sorted_e] + jnp.arange(N) - raw_off[sorted_e]).astype(jnp.int32)
    sorted_token = jnp.zeros((P,), jnp.int32).at[slot].set(
        (order // K).astype(jnp.int32))
    sorted_wt = jnp.zeros((P,), jnp.float32).at[slot].set(
        topk_weights.reshape(-1)[order])
    inv_slot = jnp.zeros((N,), jnp.int32).at[order].set(slot)
    s0 = inv_slot.reshape(T, K)[:, 0]
    s1 = inv_slot.reshape(T, K)[:, 1]
    block_expert = jnp.minimum(
        jnp.searchsorted(pad_cum, jnp.arange(NB) * BT, side="right"), E - 1
    ).astype(jnp.int32)

    sc_dispatch, sc_finalize = _build_sc_kernels()
    x_sorted = sc_dispatch(hidden_states, sorted_token.reshape(NW, NCH, CH))
    y_sorted = _tc_moe(block_expert, x_sorted, w1, w2, sorted_wt.reshape(P, 1))
    out = sc_finalize(y_sorted, s0.reshape(NW, NFCH, FCH),
                      s1.reshape(NW, NFCH, FCH))
    return out


# speedup vs baseline: 1.1534x; 1.0642x over previous
import functools

import jax
import jax.numpy as jnp
from jax import lax
from jax.experimental import pallas as pl
from jax.experimental.pallas import tpu as pltpu
from jax.experimental.pallas import tpu_sc as plsc

E = 8
K = 2
T = 2048
D = 1024
FF = 2048

BT = 256
N = T * K
NB = (N + E * (BT - 1) + BT - 1) // BT
P = NB * BT

NC = 2
NS = 16
NW = NC * NS

ROWS_W = P // NW
CH = 48
NCH = ROWS_W // CH

TOK_W = T // NW
FCH = 32
NFCH = TOK_W // FCH

@functools.cache
def _build_sc_kernels():
    mesh = plsc.VectorSubcoreMesh(core_axis_name="c", subcore_axis_name="s")

    @functools.partial(
        pl.kernel,
        out_type=jax.ShapeDtypeStruct((P, D), jnp.float32),
        mesh=mesh,
        scratch_types=[
            pltpu.VMEM((NCH, CH), jnp.int32),
            pltpu.VMEM((2, CH, D), jnp.float32),
            pltpu.SemaphoreType.DMA,
            pltpu.SemaphoreType.DMA,
        ],
    )
    def sc_dispatch(hs_hbm, tok_hbm, out_hbm, idx_v, rows_v, sem_g, sem_w):
        wid = lax.axis_index("s") * NC + lax.axis_index("c")
        base = wid * ROWS_W
        pltpu.sync_copy(tok_hbm.at[wid], idx_v)
        gathers = [None] * NCH
        writes = [None] * NCH
        gathers[0] = pltpu.async_copy(hs_hbm.at[idx_v.at[0]], rows_v.at[0], sem_g)
        for c in range(NCH):
            gathers[c].wait()
            if c + 1 < NCH:
                if c >= 1:
                    writes[c - 1].wait()
                gathers[c + 1] = pltpu.async_copy(
                    hs_hbm.at[idx_v.at[c + 1]], rows_v.at[(c + 1) % 2], sem_g)
            writes[c] = pltpu.async_copy(
                rows_v.at[c % 2], out_hbm.at[pl.ds(base + c * CH, CH)], sem_w)
        writes[NCH - 2].wait()
        writes[NCH - 1].wait()

    @functools.partial(
        pl.kernel,
        out_type=jax.ShapeDtypeStruct((T, D), jnp.float32),
        mesh=mesh,
        scratch_types=[
            pltpu.VMEM((NFCH, 2 * FCH), jnp.int32),
            pltpu.VMEM((2 * FCH, D), jnp.float32),
            pltpu.VMEM((FCH, D), jnp.float32),
            pltpu.SemaphoreType.DMA,
        ],
    )
    def sc_finalize(y_hbm, sint_hbm, out_hbm, idx_v, g_v, o_v, sem):
        wid = lax.axis_index("s") * NC + lax.axis_index("c")
        base = wid * TOK_W
        pltpu.sync_copy(sint_hbm.at[wid], idx_v)
        for c in range(NFCH):
            pltpu.async_copy(y_hbm.at[idx_v.at[c]], g_v, sem).wait()

            def _add(j, _):
                col = j * 16
                for r in range(FCH):
                    o_v[r, pl.ds(col, 16)] = (g_v[2 * r, pl.ds(col, 16)]
                                              + g_v[2 * r + 1, pl.ds(col, 16)])
                return 0

            lax.fori_loop(0, D // 16, _add, 0)
            pltpu.sync_copy(o_v, out_hbm.at[pl.ds(base + c * FCH, FCH)])

    return sc_dispatch, sc_finalize


def _tc_moe_body(be_ref, nv_ref, x_ref, w1_ref, w2_ref, sw_ref, y_ref):
    @pl.when(pl.program_id(0) < nv_ref[0])
    def _():
        x = x_ref[...]
        w1b = w1_ref[0]
        g = lax.dot_general(x, w1b[:FF], (((1,), (1,)), ((), ())),
                            preferred_element_type=jnp.float32)
        u = lax.dot_general(x, w1b[FF:], (((1,), (1,)), ((), ())),
                            preferred_element_type=jnp.float32)
        h = g * lax.logistic(g) * u
        y = lax.dot_general(h, w2_ref[0], (((1,), (1,)), ((), ())),
                            preferred_element_type=jnp.float32)
        y_ref[...] = y * sw_ref[...]


_tc_moe = pl.pallas_call(
    _tc_moe_body,
    grid_spec=pltpu.PrefetchScalarGridSpec(
        num_scalar_prefetch=2,
        grid=(NB,),
        in_specs=[
            pl.BlockSpec((BT, D), lambda i, be, nv: (i, 0)),
            pl.BlockSpec((1, 2 * FF, D), lambda i, be, nv: (be[i], 0, 0)),
            pl.BlockSpec((1, D, FF), lambda i, be, nv: (be[i], 0, 0)),
            pl.BlockSpec((BT, 1), lambda i, be, nv: (i, 0)),
        ],
        out_specs=pl.BlockSpec((BT, D), lambda i, be, nv: (i, 0)),
    ),
    out_shape=jax.ShapeDtypeStruct((P, D), jnp.float32),
)


def kernel(hidden_states, w1, w2, topk_weights, topk_ids):
    flat_e = topk_ids.reshape(-1).astype(jnp.int32)
    order = jnp.argsort(flat_e, stable=True)
    sorted_e = flat_e[order]
    counts = jnp.bincount(flat_e, length=E)
    raw_cum = jnp.cumsum(counts)
    raw_off = raw_cum - counts
    pad_counts = ((counts + BT - 1) // BT) * BT
    pad_cum = jnp.cumsum(pad_counts)
    pad_off = pad_cum - pad_counts
    slot = (pad_off[sorted_e] + jnp.arange(N) - raw_off[sorted_e]).astype(jnp.int32)
    sorted_token = jnp.zeros((P,), jnp.int32).at[slot].set(
        (order // K).astype(jnp.int32))
    sorted_wt = jnp.zeros((P,), jnp.float32).at[slot].set(
        topk_weights.reshape(-1)[order])
    inv_slot = jnp.zeros((N,), jnp.int32).at[order].set(slot)
    sint = inv_slot.reshape(T, K)
    n_valid = (jnp.sum(pad_counts) // BT).astype(jnp.int32)
    be_raw = jnp.minimum(
        jnp.searchsorted(pad_cum, jnp.arange(NB) * BT, side="right"), E - 1
    ).astype(jnp.int32)
    block_expert = jnp.where(jnp.arange(NB) < n_valid, be_raw,
                             be_raw[jnp.maximum(n_valid - 1, 0)])

    sc_dispatch, sc_finalize = _build_sc_kernels()
    x_sorted = sc_dispatch(hidden_states, sorted_token.reshape(NW, NCH, CH))
    y_sorted = _tc_moe(block_expert, n_valid.reshape(1), x_sorted, w1, w2,
                       sorted_wt.reshape(P, 1))
    out = sc_finalize(y_sorted, sint.reshape(NW, NFCH, 2 * FCH))
    return out
